# baseline (speedup 1.0000x reference)
import functools

import jax
import jax.numpy as jnp
from jax import lax
from jax.experimental import pallas as pl
from jax.experimental.pallas import tpu as pltpu
from jax.experimental.pallas import tpu_sc as plsc

EMBEDDING_DIM = 64

_NUM_CORES = 2
_NUM_SUBCORES = 16
_NUM_WORKERS = _NUM_CORES * _NUM_SUBCORES

_CHUNK = 256
_NBUF = 3


@functools.partial(jax.jit, static_argnames=("num_pairs",))
def _embedding_gather(weight, ids_even, ids_odd, *, num_pairs):
    p_per_w = num_pairs // _NUM_WORKERS
    n_chunks = p_per_w // _CHUNK
    mesh = plsc.VectorSubcoreMesh(core_axis_name="c", subcore_axis_name="s")

    @functools.partial(
        pl.kernel,
        mesh=mesh,
        compiler_params=pltpu.CompilerParams(use_tc_tiling_on_sc=False),
        out_type=jax.ShapeDtypeStruct((num_pairs, 2 * EMBEDDING_DIM), jnp.float32),
        scratch_types=[
            pltpu.VMEM((p_per_w,), jnp.int32),
            pltpu.VMEM((p_per_w,), jnp.int32),
            *[pltpu.VMEM((_CHUNK, EMBEDDING_DIM), jnp.float32) for _ in range(2 * _NBUF)],
            *[pltpu.SemaphoreType.DMA for _ in range(4 * _NBUF)],
        ],
    )
    def gather_kernel(table_hbm, ide_hbm, ido_hbm, out_hbm, ide_v, ido_v, *bufs_and_sems):
        rows_e = bufs_and_sems[:_NBUF]
        rows_o = bufs_and_sems[_NBUF : 2 * _NBUF]
        gsem_e = bufs_and_sems[2 * _NBUF : 3 * _NBUF]
        gsem_o = bufs_and_sems[3 * _NBUF : 4 * _NBUF]
        osem_e = bufs_and_sems[4 * _NBUF : 5 * _NBUF]
        osem_o = bufs_and_sems[5 * _NBUF : 6 * _NBUF]

        wid = lax.axis_index("s") * _NUM_CORES + lax.axis_index("c")
        base = wid * p_per_w

        pltpu.sync_copy(ide_hbm.at[pl.ds(base, p_per_w)], ide_v)
        pltpu.sync_copy(ido_hbm.at[pl.ds(base, p_per_w)], ido_v)

        def start_gathers(g):
            b = g % _NBUF
            sl = pl.ds(g * _CHUNK, _CHUNK)
            return (
                pltpu.async_copy(table_hbm.at[ide_v.at[sl]], rows_e[b], gsem_e[b]),
                pltpu.async_copy(table_hbm.at[ido_v.at[sl]], rows_o[b], gsem_o[b]),
            )

        def start_outs(g):
            b = g % _NBUF
            rsl = pl.ds(base + g * _CHUNK, _CHUNK)
            return (
                pltpu.async_copy(
                    rows_e[b], out_hbm.at[rsl, pl.ds(0, EMBEDDING_DIM)], osem_e[b]
                ),
                pltpu.async_copy(
                    rows_o[b],
                    out_hbm.at[rsl, pl.ds(EMBEDDING_DIM, EMBEDDING_DIM)],
                    osem_o[b],
                ),
            )

        gathers = [None] * n_chunks
        outs = [None] * n_chunks
        gathers[0] = start_gathers(0)
        if n_chunks > 1:
            gathers[1] = start_gathers(1)
        for g in range(n_chunks):
            for h in gathers[g]:
                h.wait()
            outs[g] = start_outs(g)
            g2 = g + 2
            if g2 < n_chunks:
                if g2 >= _NBUF:
                    for h in outs[g2 - _NBUF]:
                        h.wait()
                gathers[g2] = start_gathers(g2)
        for g in range(max(0, n_chunks - _NBUF), n_chunks):
            for h in outs[g]:
                h.wait()

    return gather_kernel(weight, ids_even, ids_odd)


def kernel(token_ids, weight):
    batch, seq = token_ids.shape
    dim = weight.shape[1]
    flat = token_ids.reshape(-1).astype(jnp.int32)
    out128 = _embedding_gather(
        weight, flat[0::2], flat[1::2], num_pairs=(batch * seq) // 2
    )
    return out128.reshape(batch, seq, dim)
